# baseline (speedup 1.0000x reference)
import functools

import jax
import jax.numpy as jnp
from jax import lax
from jax.experimental import pallas as pl
from jax.experimental.pallas import tpu as pltpu
from jax.experimental.pallas import tpu_sc as plsc

N_NODES = 100000
D = 128
S = 256
NC = 2
NS = 16
NW = NC * NS
CHUNK = 3136
NPAD = NW * CHUNK
BLK = 224
NBLK = CHUNK // BLK
NGRP = BLK // 16
SP = S + 1
R = D // 16
TAIL0 = (NW - 1) * CHUNK


def _accumulate_body(feat_hbm, tail_hbm, seg_hbm, wg_hbm, bg_hbm,
                     pacc_hbm, pd_hbm,
                     fbuf, ids_v, wg_v, bg_v, acc_v, d_v, sem0, sem1):
    w = lax.axis_index("s") * NC + lax.axis_index("c")
    row0 = w * CHUNK
    is_main = w < NW - 1

    def fill(buf, sem, b):
        @pl.when(is_main)
        def _():
            pltpu.async_copy(feat_hbm.at[pl.ds(row0 + b * BLK, BLK), :],
                             buf, sem)

        @pl.when(jnp.logical_not(is_main))
        def _():
            pltpu.async_copy(tail_hbm.at[pl.ds(b * BLK, BLK), :], buf, sem)

    fill(fbuf.at[0], sem0, 0)
    fill(fbuf.at[1], sem1, 1)

    pltpu.sync_copy(seg_hbm.at[pl.ds(row0, CHUNK)], ids_v)
    pltpu.sync_copy(wg_hbm, wg_v)
    pltpu.sync_copy(bg_hbm, bg_v)

    zeros16 = jnp.zeros((16,), jnp.float32)

    @plsc.parallel_loop(0, SP, 1, unroll=4)
    def zero_body(i):
        for r in range(R):
            acc_v[i, pl.ds(r * 16, 16)] = zeros16
        d_v[i, :] = zeros16

    wgv = [wg_v[pl.ds(r * 16, 16)] for r in range(R)]
    bgv = bg_v[:]
    lane = lax.iota(jnp.int32, 16)
    dnums = lax.GatherDimensionNumbers(
        offset_dims=(), collapsed_slice_dims=(0,), start_index_map=(0,))

    def vperm(v, idx):
        return lax.gather(
            v, idx[:, None], dimension_numbers=dnums, slice_sizes=(1,),
            unique_indices=True, indices_are_sorted=False,
            mode=lax.GatherScatterMode.PROMISE_IN_BOUNDS)

    xor_idx = {k: jnp.bitwise_xor(lane, k) for k in (1, 2, 4, 8)}
    masks = {k: (lane & k) == 0 for k in (1, 2)}

    def combine(a, b, k):
        m = masks[k]
        return (jnp.where(m, a, vperm(b, xor_idx[k]))
                + jnp.where(m, vperm(a, xor_idx[k]), b))

    def tree_sum(vs):
        while len(vs) > 1:
            vs = [x + y for x, y in zip(vs[::2], vs[1::2])]
        return vs[0]

    def process(buf, b):
        def grp_body(g, carry):
            ids16 = ids_v[pl.ds(b * BLK + g * 16, 16)]
            for q in range(8):
                base = g * 16 + q * 2
                frs = []
                ps = []
                for j in range(2):
                    fr = [buf[base + j, pl.ds(r * 16, 16)] for r in range(R)]
                    frs.append(fr)
                    ps.append(tree_sum([fr[r] * wgv[r] for r in range(R)]))
                c = combine(ps[0], ps[1], 1)
                c = c + vperm(c, xor_idx[2])
                c = c + vperm(c, xor_idx[4])
                c = c + vperm(c, xor_idx[8])
                epack = jnp.exp(c + bgv)
                for j in range(2):
                    ls = ids16[q * 2 + j]
                    ev = jnp.broadcast_to(epack[j], (16,))
                    plsc.addupdate(d_v.at[ls, :], ev)
                    for r in range(R):
                        plsc.addupdate(acc_v.at[ls, pl.ds(r * 16, 16)],
                                       ev * frs[j][r])
            return carry

        lax.fori_loop(0, NGRP, grp_body, None)

    def wait_fill(buf, sem):
        pltpu.make_async_copy(feat_hbm.at[pl.ds(0, BLK), :], buf, sem).wait()

    def pair_body(h, carry):
        b0 = 2 * h
        b1 = b0 + 1
        wait_fill(fbuf.at[0], sem0)
        process(fbuf.at[0], b0)

        @pl.when(b0 + 2 < NBLK)
        def _():
            fill(fbuf.at[0], sem0, b0 + 2)

        wait_fill(fbuf.at[1], sem1)
        process(fbuf.at[1], b1)

        @pl.when(b1 + 2 < NBLK)
        def _():
            fill(fbuf.at[1], sem1, b1 + 2)

        return carry

    lax.fori_loop(0, NBLK // 2, pair_body, None)

    pltpu.sync_copy(acc_v, pacc_hbm.at[w])
    pltpu.sync_copy(d_v, pd_hbm.at[w])


def _merge_body(pacc_hbm, pd_hbm, out_hbm, abuf, dbuf, obuf, dacc):
    w = lax.axis_index("s") * NC + lax.axis_index("c")
    nseg = S // NW
    s0 = w * nseg

    pltpu.sync_copy(pacc_hbm.at[:, pl.ds(s0, nseg), :], abuf)
    pltpu.sync_copy(pd_hbm.at[:, pl.ds(s0, nseg), :], dbuf)

    zeros16 = jnp.zeros((16,), jnp.float32)
    for j in range(nseg):
        dacc[j, :] = zeros16
        for r in range(R):
            obuf[j, pl.ds(r * 16, 16)] = zeros16

    @plsc.parallel_loop(0, NW, 1, unroll=4)
    def kbody(k):
        for j in range(nseg):
            plsc.addupdate(dacc.at[j, :], dbuf[k, j, :])
            for r in range(R):
                sl = pl.ds(r * 16, 16)
                plsc.addupdate(obuf.at[j, sl], abuf[k, j, sl])

    for j in range(nseg):
        recip = 1.0 / (dacc[j, :] + 1e-12)
        for r in range(R):
            sl = pl.ds(r * 16, 16)
            obuf[j, sl] = obuf[j, sl] * recip

    pltpu.sync_copy(obuf, out_hbm.at[pl.ds(s0, nseg), :])


_MESH = plsc.VectorSubcoreMesh(core_axis_name="c", subcore_axis_name="s")

_accumulate = functools.partial(
    pl.kernel,
    out_type=(
        jax.ShapeDtypeStruct((NW, SP, D), jnp.float32),
        jax.ShapeDtypeStruct((NW, SP, 16), jnp.float32),
    ),
    mesh=_MESH,
    scratch_types=[
        pltpu.VMEM((2, BLK, D), jnp.float32),
        pltpu.VMEM((CHUNK,), jnp.int32),
        pltpu.VMEM((D,), jnp.float32),
        pltpu.VMEM((16,), jnp.float32),
        pltpu.VMEM((SP, D), jnp.float32),
        pltpu.VMEM((SP, 16), jnp.float32),
        pltpu.SemaphoreType.DMA,
        pltpu.SemaphoreType.DMA,
    ],
)(_accumulate_body)

_merge = functools.partial(
    pl.kernel,
    out_type=jax.ShapeDtypeStruct((S, D), jnp.float32),
    mesh=_MESH,
    scratch_types=[
        pltpu.VMEM((NW, S // NW, D), jnp.float32),
        pltpu.VMEM((NW, S // NW, 16), jnp.float32),
        pltpu.VMEM((S // NW, D), jnp.float32),
        pltpu.VMEM((S // NW, 16), jnp.float32),
    ],
)(_merge_body)


def kernel(feat, segment_ids, Wg, bg):
    tail = jnp.zeros((CHUNK, D), jnp.float32).at[:N_NODES - TAIL0].set(
        feat[TAIL0:])
    seg_p = jnp.pad(segment_ids.astype(jnp.int32), (0, NPAD - N_NODES),
                    constant_values=S)
    wg = Wg.reshape(D)
    bgv = jnp.broadcast_to(bg.astype(jnp.float32), (16,))
    pacc, pd = _accumulate(feat, tail, seg_p, wg, bgv)
    return _merge(pacc, pd)


# speedup vs baseline: 1.2880x; 1.2880x over previous
import functools

import jax
import jax.numpy as jnp
from jax import lax
from jax.experimental import pallas as pl
from jax.experimental.pallas import tpu as pltpu
from jax.experimental.pallas import tpu_sc as plsc

N_NODES = 100000
D = 128
S = 256
NC = 2
NS = 16
NW = NC * NS
CHUNK = 3136
NPAD = NW * CHUNK
BLK = 224
NBLK = CHUNK // BLK
NGRP = BLK // 16
SP = S + 1
R = D // 16
TAIL0 = (NW - 1) * CHUNK


def _accumulate_body(feat_hbm, tail_hbm, seg_hbm, wg_hbm, bg_hbm,
                     pacc_hbm, pd_hbm,
                     fbuf, ids_v, wg_v, bg_v, acc_v, d_v, sem0, sem1):
    w = lax.axis_index("s") * NC + lax.axis_index("c")
    row0 = w * CHUNK
    is_main = w < NW - 1

    def fill(buf, sem, b):
        @pl.when(is_main)
        def _():
            pltpu.async_copy(feat_hbm.at[pl.ds(row0 + b * BLK, BLK), :],
                             buf, sem)

        @pl.when(jnp.logical_not(is_main))
        def _():
            pltpu.async_copy(tail_hbm.at[pl.ds(b * BLK, BLK), :], buf, sem)

    fill(fbuf.at[0], sem0, 0)
    fill(fbuf.at[1], sem1, 1)

    pltpu.sync_copy(seg_hbm.at[pl.ds(row0, CHUNK)], ids_v)
    pltpu.sync_copy(wg_hbm, wg_v)
    pltpu.sync_copy(bg_hbm, bg_v)

    zeros16 = jnp.zeros((16,), jnp.float32)

    @plsc.parallel_loop(0, SP, 1, unroll=4)
    def zero_body(i):
        for r in range(R):
            acc_v[i, pl.ds(r * 16, 16)] = zeros16
        d_v[i, :] = zeros16

    wgv = [wg_v[pl.ds(r * 16, 16)] for r in range(R)]
    bgv = bg_v[:]
    lane = lax.iota(jnp.int32, 16)
    dnums = lax.GatherDimensionNumbers(
        offset_dims=(), collapsed_slice_dims=(0,), start_index_map=(0,))

    def vperm(v, idx):
        return lax.gather(
            v, idx[:, None], dimension_numbers=dnums, slice_sizes=(1,),
            unique_indices=True, indices_are_sorted=False,
            mode=lax.GatherScatterMode.PROMISE_IN_BOUNDS)

    xor_idx = {k: jnp.bitwise_xor(lane, k) for k in (1, 2, 4, 8)}
    masks = {k: (lane & k) == 0 for k in (1, 2)}

    def combine(a, b, k):
        m = masks[k]
        return (jnp.where(m, a, vperm(b, xor_idx[k]))
                + jnp.where(m, vperm(a, xor_idx[k]), b))

    def tree_sum(vs):
        while len(vs) > 1:
            vs = [x + y for x, y in zip(vs[::2], vs[1::2])]
        return vs[0]

    def process(buf, b):
        def grp_body(g, carry):
            ids16 = ids_v[pl.ds(b * BLK + g * 16, 16)]

            def gate_pack(q):
                base = g * 16 + q * 4
                frs = []
                ps = []
                for j in range(4):
                    fr = [buf[base + j, pl.ds(r * 16, 16)] for r in range(R)]
                    frs.append(fr)
                    ps.append(tree_sum([fr[r] * wgv[r] for r in range(R)]))
                c = combine(combine(ps[0], ps[1], 1),
                            combine(ps[2], ps[3], 1), 2)
                c = c + vperm(c, xor_idx[4])
                c = c + vperm(c, xor_idx[8])
                return frs, jnp.exp(c + bgv)

            def uniform_grp(_):
                ls = ids16[0]
                wsum = None
                dsum = None
                for q in range(4):
                    frs, epack = gate_pack(q)
                    dsum = epack if dsum is None else dsum + epack
                    for j in range(4):
                        ev = jnp.broadcast_to(epack[j], (16,))
                        if wsum is None:
                            wsum = [ev * frs[j][r] for r in range(R)]
                        else:
                            wsum = [wsum[r] + ev * frs[j][r]
                                    for r in range(R)]
                plsc.addupdate(d_v.at[ls, :], dsum)
                for r in range(R):
                    plsc.addupdate(acc_v.at[ls, pl.ds(r * 16, 16)], wsum[r])
                return 0

            def generic_grp(_):
                for q in range(4):
                    frs, epack = gate_pack(q)
                    epack4 = epack * 0.25
                    for j in range(4):
                        ls = ids16[q * 4 + j]
                        ev = jnp.broadcast_to(epack[j], (16,))
                        plsc.addupdate(d_v.at[ls, :],
                                       jnp.broadcast_to(epack4[j], (16,)))
                        for r in range(R):
                            plsc.addupdate(acc_v.at[ls, pl.ds(r * 16, 16)],
                                           ev * frs[j][r])
                return 0

            lax.cond(ids16[0] == ids16[15], uniform_grp, generic_grp, 0)
            return carry

        lax.fori_loop(0, NGRP, grp_body, None)

    def wait_fill(buf, sem):
        pltpu.make_async_copy(feat_hbm.at[pl.ds(0, BLK), :], buf, sem).wait()

    def pair_body(h, carry):
        b0 = 2 * h
        b1 = b0 + 1
        wait_fill(fbuf.at[0], sem0)
        process(fbuf.at[0], b0)

        @pl.when(b0 + 2 < NBLK)
        def _():
            fill(fbuf.at[0], sem0, b0 + 2)

        wait_fill(fbuf.at[1], sem1)
        process(fbuf.at[1], b1)

        @pl.when(b1 + 2 < NBLK)
        def _():
            fill(fbuf.at[1], sem1, b1 + 2)

        return carry

    lax.fori_loop(0, NBLK // 2, pair_body, None)

    pltpu.sync_copy(acc_v, pacc_hbm.at[w])
    pltpu.sync_copy(d_v, pd_hbm.at[w])


def _merge_body(pacc_hbm, pd_hbm, out_hbm, abuf, dbuf, obuf, dacc):
    w = lax.axis_index("s") * NC + lax.axis_index("c")
    nseg = S // NW
    s0 = w * nseg
    lane = lax.iota(jnp.int32, 16)
    dnums = lax.GatherDimensionNumbers(
        offset_dims=(), collapsed_slice_dims=(0,), start_index_map=(0,))

    def vperm(v, idx):
        return lax.gather(
            v, idx[:, None], dimension_numbers=dnums, slice_sizes=(1,),
            unique_indices=True, indices_are_sorted=False,
            mode=lax.GatherScatterMode.PROMISE_IN_BOUNDS)

    xor_idx = {k: jnp.bitwise_xor(lane, k) for k in (1, 2, 4, 8)}

    pltpu.sync_copy(pacc_hbm.at[:, pl.ds(s0, nseg), :], abuf)
    pltpu.sync_copy(pd_hbm.at[:, pl.ds(s0, nseg), :], dbuf)

    zeros16 = jnp.zeros((16,), jnp.float32)
    for j in range(nseg):
        dacc[j, :] = zeros16
        for r in range(R):
            obuf[j, pl.ds(r * 16, 16)] = zeros16

    @plsc.parallel_loop(0, NW, 1, unroll=4)
    def kbody(k):
        for j in range(nseg):
            plsc.addupdate(dacc.at[j, :], dbuf[k, j, :])
            for r in range(R):
                sl = pl.ds(r * 16, 16)
                plsc.addupdate(obuf.at[j, sl], abuf[k, j, sl])

    for j in range(nseg):
        ds = dacc[j, :]
        for k in (1, 2, 4, 8):
            ds = ds + vperm(ds, xor_idx[k])
        recip = 4.0 / (ds + 4e-12)
        for r in range(R):
            sl = pl.ds(r * 16, 16)
            obuf[j, sl] = obuf[j, sl] * recip

    pltpu.sync_copy(obuf, out_hbm.at[pl.ds(s0, nseg), :])


_MESH = plsc.VectorSubcoreMesh(core_axis_name="c", subcore_axis_name="s")

_accumulate = functools.partial(
    pl.kernel,
    out_type=(
        jax.ShapeDtypeStruct((NW, SP, D), jnp.float32),
        jax.ShapeDtypeStruct((NW, SP, 16), jnp.float32),
    ),
    mesh=_MESH,
    scratch_types=[
        pltpu.VMEM((2, BLK, D), jnp.float32),
        pltpu.VMEM((CHUNK,), jnp.int32),
        pltpu.VMEM((D,), jnp.float32),
        pltpu.VMEM((16,), jnp.float32),
        pltpu.VMEM((SP, D), jnp.float32),
        pltpu.VMEM((SP, 16), jnp.float32),
        pltpu.SemaphoreType.DMA,
        pltpu.SemaphoreType.DMA,
    ],
)(_accumulate_body)

_merge = functools.partial(
    pl.kernel,
    out_type=jax.ShapeDtypeStruct((S, D), jnp.float32),
    mesh=_MESH,
    scratch_types=[
        pltpu.VMEM((NW, S // NW, D), jnp.float32),
        pltpu.VMEM((NW, S // NW, 16), jnp.float32),
        pltpu.VMEM((S // NW, D), jnp.float32),
        pltpu.VMEM((S // NW, 16), jnp.float32),
    ],
)(_merge_body)


def kernel(feat, segment_ids, Wg, bg):
    tail = jnp.zeros((CHUNK, D), jnp.float32).at[:N_NODES - TAIL0].set(
        feat[TAIL0:])
    seg_p = jnp.pad(segment_ids.astype(jnp.int32), (0, NPAD - N_NODES),
                    constant_values=S)
    wg = Wg.reshape(D)
    bgv = jnp.broadcast_to(bg.astype(jnp.float32), (16,))
    pacc, pd = _accumulate(feat, tail, seg_p, wg, bgv)
    return _merge(pacc, pd)


# speedup vs baseline: 1.3056x; 1.0137x over previous
import functools

import jax
import jax.numpy as jnp
from jax import lax
from jax.experimental import pallas as pl
from jax.experimental.pallas import tpu as pltpu
from jax.experimental.pallas import tpu_sc as plsc

N_NODES = 100000
D = 128
S = 256
NC = 2
NS = 16
NW = NC * NS
CHUNK = 3136
NPAD = NW * CHUNK
BLK = 112
NBLK = CHUNK // BLK
NGRP = BLK // 16
SP = S + 1
R = D // 16
TAIL0 = (NW - 1) * CHUNK


def _accumulate_body(feat_hbm, tail_hbm, seg_hbm, wg_hbm, bg_hbm,
                     pacc_hbm, pd_hbm,
                     fbuf, ids_v, wg_v, bg_v, acc_v, d_v, sem0, sem1):
    w = lax.axis_index("s") * NC + lax.axis_index("c")
    row0 = w * CHUNK
    is_main = w < NW - 1

    def fill(buf, sem, b):
        @pl.when(is_main)
        def _():
            pltpu.async_copy(feat_hbm.at[pl.ds(row0 + b * BLK, BLK), :],
                             buf, sem)

        @pl.when(jnp.logical_not(is_main))
        def _():
            pltpu.async_copy(tail_hbm.at[pl.ds(b * BLK, BLK), :], buf, sem)

    fill(fbuf.at[0], sem0, 0)
    fill(fbuf.at[1], sem1, 1)

    pltpu.sync_copy(seg_hbm.at[pl.ds(row0, CHUNK)], ids_v)
    pltpu.sync_copy(wg_hbm, wg_v)
    pltpu.sync_copy(bg_hbm, bg_v)

    zeros16 = jnp.zeros((16,), jnp.float32)

    @plsc.parallel_loop(0, SP, 1, unroll=4)
    def zero_body(i):
        for r in range(R):
            acc_v[i, pl.ds(r * 16, 16)] = zeros16
        d_v[i, :] = zeros16

    wgv = [wg_v[pl.ds(r * 16, 16)] for r in range(R)]
    bgv = bg_v[:]
    lane = lax.iota(jnp.int32, 16)
    dnums = lax.GatherDimensionNumbers(
        offset_dims=(), collapsed_slice_dims=(0,), start_index_map=(0,))

    def vperm(v, idx):
        return lax.gather(
            v, idx[:, None], dimension_numbers=dnums, slice_sizes=(1,),
            unique_indices=True, indices_are_sorted=False,
            mode=lax.GatherScatterMode.PROMISE_IN_BOUNDS)

    xor_idx = {k: jnp.bitwise_xor(lane, k) for k in (1, 2, 4, 8)}
    masks = {k: (lane & k) == 0 for k in (1, 2)}

    def combine(a, b, k):
        m = masks[k]
        return (jnp.where(m, a, vperm(b, xor_idx[k]))
                + jnp.where(m, vperm(a, xor_idx[k]), b))

    def tree_sum(vs):
        while len(vs) > 1:
            vs = [x + y for x, y in zip(vs[::2], vs[1::2])]
        return vs[0]

    def process(buf, b):
        def grp_body(g, carry):
            ids16 = ids_v[pl.ds(b * BLK + g * 16, 16)]

            def gate_pack(q):
                base = g * 16 + q * 4
                frs = []
                ps = []
                for j in range(4):
                    fr = [buf[base + j, pl.ds(r * 16, 16)] for r in range(R)]
                    frs.append(fr)
                    ps.append(tree_sum([fr[r] * wgv[r] for r in range(R)]))
                c = combine(combine(ps[0], ps[1], 1),
                            combine(ps[2], ps[3], 1), 2)
                c = c + vperm(c, xor_idx[4])
                c = c + vperm(c, xor_idx[8])
                return frs, jnp.exp(c + bgv)

            def uniform_grp(_):
                ls = ids16[0]
                wsum = None
                dsum = None
                for q in range(4):
                    frs, epack = gate_pack(q)
                    dsum = epack if dsum is None else dsum + epack
                    for j in range(4):
                        ev = jnp.broadcast_to(epack[j], (16,))
                        if wsum is None:
                            wsum = [ev * frs[j][r] for r in range(R)]
                        else:
                            wsum = [wsum[r] + ev * frs[j][r]
                                    for r in range(R)]
                plsc.addupdate(d_v.at[ls, :], dsum)
                for r in range(R):
                    plsc.addupdate(acc_v.at[ls, pl.ds(r * 16, 16)], wsum[r])
                return 0

            def generic_grp(_):
                for q in range(4):
                    frs, epack = gate_pack(q)
                    epack4 = epack * 0.25
                    for j in range(4):
                        ls = ids16[q * 4 + j]
                        ev = jnp.broadcast_to(epack[j], (16,))
                        plsc.addupdate(d_v.at[ls, :],
                                       jnp.broadcast_to(epack4[j], (16,)))
                        for r in range(R):
                            plsc.addupdate(acc_v.at[ls, pl.ds(r * 16, 16)],
                                           ev * frs[j][r])
                return 0

            lax.cond(ids16[0] == ids16[15], uniform_grp, generic_grp, 0)
            return carry

        lax.fori_loop(0, NGRP, grp_body, None)

    def wait_fill(buf, sem):
        pltpu.make_async_copy(feat_hbm.at[pl.ds(0, BLK), :], buf, sem).wait()

    def pair_body(h, carry):
        b0 = 2 * h
        b1 = b0 + 1
        wait_fill(fbuf.at[0], sem0)
        process(fbuf.at[0], b0)

        @pl.when(b0 + 2 < NBLK)
        def _():
            fill(fbuf.at[0], sem0, b0 + 2)

        wait_fill(fbuf.at[1], sem1)
        process(fbuf.at[1], b1)

        @pl.when(b1 + 2 < NBLK)
        def _():
            fill(fbuf.at[1], sem1, b1 + 2)

        return carry

    lax.fori_loop(0, NBLK // 2, pair_body, None)

    pltpu.sync_copy(acc_v, pacc_hbm.at[w])
    pltpu.sync_copy(d_v, pd_hbm.at[w])


def _merge_body(pacc_hbm, pd_hbm, out_hbm, abuf, dbuf, obuf, dacc):
    w = lax.axis_index("s") * NC + lax.axis_index("c")
    nseg = S // NW
    s0 = w * nseg
    lane = lax.iota(jnp.int32, 16)
    dnums = lax.GatherDimensionNumbers(
        offset_dims=(), collapsed_slice_dims=(0,), start_index_map=(0,))

    def vperm(v, idx):
        return lax.gather(
            v, idx[:, None], dimension_numbers=dnums, slice_sizes=(1,),
            unique_indices=True, indices_are_sorted=False,
            mode=lax.GatherScatterMode.PROMISE_IN_BOUNDS)

    xor_idx = {k: jnp.bitwise_xor(lane, k) for k in (1, 2, 4, 8)}

    pltpu.sync_copy(pacc_hbm.at[:, pl.ds(s0, nseg), :], abuf)
    pltpu.sync_copy(pd_hbm.at[:, pl.ds(s0, nseg), :], dbuf)

    zeros16 = jnp.zeros((16,), jnp.float32)
    for j in range(nseg):
        dacc[j, :] = zeros16
        for r in range(R):
            obuf[j, pl.ds(r * 16, 16)] = zeros16

    @plsc.parallel_loop(0, NW, 1, unroll=4)
    def kbody(k):
        for j in range(nseg):
            plsc.addupdate(dacc.at[j, :], dbuf[k, j, :])
            for r in range(R):
                sl = pl.ds(r * 16, 16)
                plsc.addupdate(obuf.at[j, sl], abuf[k, j, sl])

    for j in range(nseg):
        ds = dacc[j, :]
        for k in (1, 2, 4, 8):
            ds = ds + vperm(ds, xor_idx[k])
        recip = 4.0 / (ds + 4e-12)
        for r in range(R):
            sl = pl.ds(r * 16, 16)
            obuf[j, sl] = obuf[j, sl] * recip

    pltpu.sync_copy(obuf, out_hbm.at[pl.ds(s0, nseg), :])


_MESH = plsc.VectorSubcoreMesh(core_axis_name="c", subcore_axis_name="s")

_accumulate = functools.partial(
    pl.kernel,
    out_type=(
        jax.ShapeDtypeStruct((NW, SP, D), jnp.float32),
        jax.ShapeDtypeStruct((NW, SP, 16), jnp.float32),
    ),
    mesh=_MESH,
    scratch_types=[
        pltpu.VMEM((2, BLK, D), jnp.float32),
        pltpu.VMEM((CHUNK,), jnp.int32),
        pltpu.VMEM((D,), jnp.float32),
        pltpu.VMEM((16,), jnp.float32),
        pltpu.VMEM((SP, D), jnp.float32),
        pltpu.VMEM((SP, 16), jnp.float32),
        pltpu.SemaphoreType.DMA,
        pltpu.SemaphoreType.DMA,
    ],
)(_accumulate_body)

_merge = functools.partial(
    pl.kernel,
    out_type=jax.ShapeDtypeStruct((S, D), jnp.float32),
    mesh=_MESH,
    scratch_types=[
        pltpu.VMEM((NW, S // NW, D), jnp.float32),
        pltpu.VMEM((NW, S // NW, 16), jnp.float32),
        pltpu.VMEM((S // NW, D), jnp.float32),
        pltpu.VMEM((S // NW, 16), jnp.float32),
    ],
)(_merge_body)


def kernel(feat, segment_ids, Wg, bg):
    tail = jnp.zeros((CHUNK, D), jnp.float32).at[:N_NODES - TAIL0].set(
        feat[TAIL0:])
    seg_p = jnp.pad(segment_ids.astype(jnp.int32), (0, NPAD - N_NODES),
                    constant_values=S)
    wg = Wg.reshape(D)
    bgv = jnp.broadcast_to(bg.astype(jnp.float32), (16,))
    pacc, pd = _accumulate(feat, tail, seg_p, wg, bgv)
    return _merge(pacc, pd)
